# baseline (speedup 1.0000x reference)
import functools

import jax
import jax.numpy as jnp
from jax import lax
from jax.experimental import pallas as pl
from jax.experimental.pallas import tpu as pltpu
from jax.experimental.pallas import tpu_sc as plsc

N = 10000
E = 160000
D = 256

NC = 2
NS = 16
NW = NC * NS
CPW = D // NW

CB = 1280
NCHUNK = E // CB
GSUB = 128
NG = CB // GSUB

ROWBLK = 1000


def _mm_dual_body(x_ref, wp_ref, bp_ref, ws_ref, bs_ref, hp_ref, s_ref):
    xb = x_ref[...]
    hp = jnp.dot(xb, wp_ref[...], preferred_element_type=jnp.float32)
    hp_ref[...] = jnp.maximum(hp + bp_ref[...], 0.0)
    s = jnp.dot(xb, ws_ref[...], preferred_element_type=jnp.float32)
    s_ref[...] = s + bs_ref[...]


def _mm_dual(h, WpT, bp, WsT, bs):
    return pl.pallas_call(
        _mm_dual_body,
        grid=(N // ROWBLK,),
        in_specs=[
            pl.BlockSpec((ROWBLK, D), lambda i: (i, 0)),
            pl.BlockSpec((D, D), lambda i: (0, 0)),
            pl.BlockSpec((1, D), lambda i: (0, 0)),
            pl.BlockSpec((D, D), lambda i: (0, 0)),
            pl.BlockSpec((1, D), lambda i: (0, 0)),
        ],
        out_specs=[
            pl.BlockSpec((ROWBLK, D), lambda i: (i, 0)),
            pl.BlockSpec((ROWBLK, D), lambda i: (i, 0)),
        ],
        out_shape=[jax.ShapeDtypeStruct((N, D), jnp.float32)] * 2,
    )(h, WpT, bp.reshape(1, D), WsT, bs.reshape(1, D))


def _mm_out_body(act, s_ref, agg_ref, wn_ref, o_ref):
    o = s_ref[...] + jnp.dot(agg_ref[...], wn_ref[...],
                             preferred_element_type=jnp.float32)
    if act:
        o = jnp.tanh(o)
    o_ref[...] = o


def _mm_out(s, agg, WnT, act):
    return pl.pallas_call(
        functools.partial(_mm_out_body, act),
        grid=(N // ROWBLK,),
        in_specs=[
            pl.BlockSpec((ROWBLK, D), lambda i: (i, 0)),
            pl.BlockSpec((ROWBLK, D), lambda i: (i, 0)),
            pl.BlockSpec((D, D), lambda i: (0, 0)),
        ],
        out_specs=pl.BlockSpec((ROWBLK, D), lambda i: (i, 0)),
        out_shape=jax.ShapeDtypeStruct((N, D), jnp.float32),
    )(s, agg, WnT)


def _segmax_body(hp8_hbm, gidx_hbm, dst_hbm, out_hbm,
                 gidx_v, dst_v, rows_v, acc_v, gsem):
    w = lax.axis_index("s") * NC + lax.axis_index("c")

    iota = lax.iota(jnp.int32, 16)
    colpat = jnp.bitwise_and(iota, 7)
    pairsel = jnp.right_shift(iota, 3)
    pairswp = 1 - pairsel
    perm8 = jnp.bitwise_xor(iota, 8)
    wvec = jnp.full((16,), 0, jnp.int32) + w
    zeros16 = jnp.zeros((16,), jnp.float32)

    def zbody(i, carry):
        acc_v[pl.ds(i * 16, 16)] = zeros16
        return carry

    lax.fori_loop(0, (N * CPW) // 16, zbody, 0)

    def chunk_body(c, carry):
        e0 = c * CB
        pltpu.sync_copy(gidx_hbm.at[pl.ds(e0, CB)], gidx_v)
        pltpu.sync_copy(dst_hbm.at[pl.ds(e0, CB)], dst_v)

        def addw_body(i, carry2):
            sl = pl.ds(i * 16, 16)
            gidx_v[sl] = gidx_v[sl] + wvec
            return carry2

        lax.fori_loop(0, CB // 16, addw_body, 0)
        cps = [
            pltpu.async_copy(hp8_hbm.at[gidx_v.at[pl.ds(j * GSUB, GSUB)]],
                             rows_v.at[pl.ds(j * GSUB, GSUB)], gsem)
            for j in range(NG)
        ]
        for cp in cps:
            cp.wait()

        def pair_body(t, carry2):
            base = t * 2
            rowpat = pairsel + base
            dvec = plsc.load_gather(dst_v, [rowpat])
            dswp = plsc.load_gather(dst_v, [pairswp + base])
            r = plsc.load_gather(rows_v, [rowpat, colpat])
            fidx = dvec * CPW + colpat
            a = plsc.load_gather(acc_v, [fidx])
            m = jnp.maximum(a, r)
            msw = m.at[perm8].get(mode="promise_in_bounds",
                                  unique_indices=True)
            msel = jnp.where(dvec == dswp, jnp.maximum(m, msw), m)
            plsc.store_scatter(acc_v, [fidx], msel)
            return carry2

        lax.fori_loop(0, CB // 2, pair_body, 0)
        return carry

    lax.fori_loop(0, NCHUNK, chunk_body, 0)
    pltpu.sync_copy(acc_v, out_hbm.at[pl.ds(w * (N * CPW), N * CPW)])


_segmax = pl.kernel(
    _segmax_body,
    out_type=jax.ShapeDtypeStruct((NW * N * CPW,), jnp.float32),
    mesh=plsc.VectorSubcoreMesh(core_axis_name="c", subcore_axis_name="s",
                                num_cores=NC, num_subcores=NS),
    scratch_types=[
        pltpu.VMEM((CB,), jnp.int32),
        pltpu.VMEM((CB,), jnp.int32),
        pltpu.VMEM((CB, CPW), jnp.float32),
        pltpu.VMEM((N * CPW,), jnp.float32),
        pltpu.SemaphoreType.DMA,
    ],
    compiler_params=pltpu.CompilerParams(needs_layout_passes=False,
                                         use_tc_tiling_on_sc=False),
)


def _sage_layer(h, gidx32, dst, WpT, bp, WsT, WnT, bs, act):
    hp, s = _mm_dual(h, WpT, bp, WsT, bs)
    hp8 = hp.reshape(N * NW, CPW)
    agg32 = _segmax(hp8, gidx32, dst)
    agg = agg32.reshape(NW, N, CPW).transpose(1, 0, 2).reshape(N, D)
    return _mm_out(s, agg, WnT, act)


def kernel(x, edge_index, W_pool1, b_pool1, W_self1, W_neigh1, bias1,
           W_pool2, b_pool2, W_self2, W_neigh2, bias2):
    src = edge_index[0]
    dst = edge_index[1]
    gidx32 = src * NW
    h = _sage_layer(x, gidx32, dst, W_pool1.T, b_pool1, W_self1.T,
                    W_neigh1.T, bias1, True)
    h = _sage_layer(h, gidx32, dst, W_pool2.T, b_pool2, W_self2.T,
                    W_neigh2.T, bias2, False)
    return h


# speedup vs baseline: 1.1068x; 1.1068x over previous
import functools

import jax
import jax.numpy as jnp
from jax import lax
from jax.experimental import pallas as pl
from jax.experimental.pallas import tpu as pltpu
from jax.experimental.pallas import tpu_sc as plsc

N = 10000
E = 160000
D = 256

NC = 2
NS = 16
NW = NC * NS
CPW = D // NW

CB = 1280
NCHUNK = E // CB
GSUB = 128
NG = CB // GSUB

ROWBLK = 1000


def _mm_dual_body(x_ref, wp_ref, bp_ref, ws_ref, bs_ref, hp_ref, s_ref):
    xb = x_ref[...]
    hp = jnp.dot(xb, wp_ref[...], preferred_element_type=jnp.float32)
    hp_ref[...] = jnp.maximum(hp + bp_ref[...], 0.0)
    s = jnp.dot(xb, ws_ref[...], preferred_element_type=jnp.float32)
    s_ref[...] = s + bs_ref[...]


def _mm_dual(h, WpT, bp, WsT, bs):
    return pl.pallas_call(
        _mm_dual_body,
        grid=(N // ROWBLK,),
        in_specs=[
            pl.BlockSpec((ROWBLK, D), lambda i: (i, 0)),
            pl.BlockSpec((D, D), lambda i: (0, 0)),
            pl.BlockSpec((1, D), lambda i: (0, 0)),
            pl.BlockSpec((D, D), lambda i: (0, 0)),
            pl.BlockSpec((1, D), lambda i: (0, 0)),
        ],
        out_specs=[
            pl.BlockSpec((ROWBLK, D), lambda i: (i, 0)),
            pl.BlockSpec((ROWBLK, D), lambda i: (i, 0)),
        ],
        out_shape=[jax.ShapeDtypeStruct((N, D), jnp.float32)] * 2,
    )(h, WpT, bp.reshape(1, D), WsT, bs.reshape(1, D))


def _mm_out_body(act, s_ref, agg_ref, wn_ref, o_ref):
    o = s_ref[...] + jnp.dot(agg_ref[...], wn_ref[...],
                             preferred_element_type=jnp.float32)
    if act:
        o = jnp.tanh(o)
    o_ref[...] = o


def _mm_out(s, agg, WnT, act):
    return pl.pallas_call(
        functools.partial(_mm_out_body, act),
        grid=(N // ROWBLK,),
        in_specs=[
            pl.BlockSpec((ROWBLK, D), lambda i: (i, 0)),
            pl.BlockSpec((ROWBLK, D), lambda i: (i, 0)),
            pl.BlockSpec((D, D), lambda i: (0, 0)),
        ],
        out_specs=pl.BlockSpec((ROWBLK, D), lambda i: (i, 0)),
        out_shape=jax.ShapeDtypeStruct((N, D), jnp.float32),
    )(s, agg, WnT)


def _segmax_body(hp8_hbm, gidx_hbm, dst8_hbm, out_hbm,
                 gidx_v, dst8_v, rows_v, acc_v, gsem):
    w = lax.axis_index("s") * NC + lax.axis_index("c")

    iota = lax.iota(jnp.int32, 16)
    colpat = jnp.bitwise_and(iota, 7)
    pairsel = jnp.right_shift(iota, 3)
    pairswp = 1 - pairsel
    perm8 = jnp.bitwise_xor(iota, 8)
    wvec = jnp.full((16,), 0, jnp.int32) + w
    zeros16 = jnp.zeros((16,), jnp.float32)

    def zbody(i, carry):
        acc_v[pl.ds(i * 16, 16)] = zeros16
        return carry

    lax.fori_loop(0, (N * CPW) // 16, zbody, 0)

    def chunk_body(c, carry):
        e0 = c * CB
        pltpu.sync_copy(gidx_hbm.at[pl.ds(e0, CB)], gidx_v)
        pltpu.sync_copy(dst8_hbm.at[pl.ds(e0, CB)], dst8_v)

        for i in range(CB // 16):
            sl = pl.ds(i * 16, 16)
            gidx_v[sl] = gidx_v[sl] + wvec
        cps = [
            pltpu.async_copy(hp8_hbm.at[gidx_v.at[pl.ds(j * GSUB, GSUB)]],
                             rows_v.at[pl.ds(j * GSUB, GSUB)], gsem)
            for j in range(NG)
        ]
        for cp in cps:
            cp.wait()

        def pair_body(t, carry2):
            base = t * 2
            rowpat = pairsel + base
            dvec8 = plsc.load_gather(dst8_v, [rowpat])
            dswp8 = plsc.load_gather(dst8_v, [pairswp + base])
            r = plsc.load_gather(rows_v, [rowpat, colpat])
            fidx = dvec8 + colpat
            a = plsc.load_gather(acc_v, [fidx])
            m = jnp.maximum(a, r)
            msw = m.at[perm8].get(mode="promise_in_bounds",
                                  unique_indices=True)
            msel = jnp.where(dvec8 == dswp8, jnp.maximum(m, msw), m)
            plsc.store_scatter(acc_v, [fidx], msel)
            return carry2

        lax.fori_loop(0, CB // 2, pair_body, 0, unroll=8)
        return carry

    lax.fori_loop(0, NCHUNK, chunk_body, 0)
    pltpu.sync_copy(acc_v, out_hbm.at[pl.ds(w * (N * CPW), N * CPW)])


_segmax = pl.kernel(
    _segmax_body,
    out_type=jax.ShapeDtypeStruct((NW * N * CPW,), jnp.float32),
    mesh=plsc.VectorSubcoreMesh(core_axis_name="c", subcore_axis_name="s",
                                num_cores=NC, num_subcores=NS),
    scratch_types=[
        pltpu.VMEM((CB,), jnp.int32),
        pltpu.VMEM((CB,), jnp.int32),
        pltpu.VMEM((CB, CPW), jnp.float32),
        pltpu.VMEM((N * CPW,), jnp.float32),
        pltpu.SemaphoreType.DMA,
    ],
    compiler_params=pltpu.CompilerParams(needs_layout_passes=False,
                                         use_tc_tiling_on_sc=False),
)


def _sage_layer(h, gidx32, dst8, WpT, bp, WsT, WnT, bs, act):
    hp, s = _mm_dual(h, WpT, bp, WsT, bs)
    hp8 = hp.reshape(N * NW, CPW)
    agg32 = _segmax(hp8, gidx32, dst8)
    agg = agg32.reshape(NW, N, CPW).transpose(1, 0, 2).reshape(N, D)
    return _mm_out(s, agg, WnT, act)


def kernel(x, edge_index, W_pool1, b_pool1, W_self1, W_neigh1, bias1,
           W_pool2, b_pool2, W_self2, W_neigh2, bias2):
    src = edge_index[0]
    dst = edge_index[1]
    gidx32 = src * NW
    dst8 = dst * CPW
    h = _sage_layer(x, gidx32, dst8, W_pool1.T, b_pool1, W_self1.T,
                    W_neigh1.T, bias1, True)
    h = _sage_layer(h, gidx32, dst8, W_pool2.T, b_pool2, W_self2.T,
                    W_neigh2.T, bias2, False)
    return h
